# baseline (speedup 1.0000x reference)
import functools

import jax
import jax.numpy as jnp
from jax import lax
from jax.experimental import pallas as pl
from jax.experimental.pallas import tpu as pltpu
from jax.experimental.pallas import tpu_sc as plsc

N = 10000
E = 320000
D = 128
H = 32
C = 10
G = 64
BN_EPS = 1e-5

NC, NS = 2, 16
NW = NC * NS
CH = 128
NCH = 80
EW = NCH * CH
E_PAD = NW * EW
N_PAD = 10112
RPT = N_PAD // NS
GR = 10
NGRP = NCH // GR


def _segment_sum_sc(p, src2d, dst2d, zrows):
    mesh = plsc.VectorSubcoreMesh(
        core_axis_name="c", subcore_axis_name="s", num_cores=NC, num_subcores=NS
    )

    @functools.partial(
        pl.kernel,
        out_type=jax.ShapeDtypeStruct((NC, N_PAD, H), jnp.float32),
        mesh=mesh,
        compiler_params=pltpu.CompilerParams(use_tc_tiling_on_sc=False),
        scratch_types=[
            pltpu.VMEM((NGRP, GR * CH), jnp.int32),
            pltpu.VMEM((NGRP, GR * CH), jnp.int32),
            pltpu.VMEM((2, GR * CH, H), jnp.float32),
            pltpu.VMEM_SHARED((N_PAD, H), jnp.float32),
            pltpu.SemaphoreType.DMA,
            pltpu.SemaphoreType.DMA,
        ],
    )
    def seg_kernel(p_hbm, src_hbm, dst_hbm, z_hbm, out_hbm,
                   src_v, dst_v, rows_v, acc, gsem, ssem):
        c = lax.axis_index("c")
        s = lax.axis_index("s")
        w = c * NS + s
        pltpu.sync_copy(z_hbm.at[pl.ds(s * RPT, RPT)],
                        acc.at[pl.ds(s * RPT, RPT)])
        pltpu.sync_copy(src_hbm.at[pl.ds(w * NGRP, NGRP)], src_v)
        pltpu.sync_copy(dst_hbm.at[pl.ds(w * NGRP, NGRP)], dst_v)
        plsc.subcore_barrier()

        pltpu.async_copy(p_hbm.at[src_v.at[0]], rows_v.at[0], gsem)

        @pl.loop(0, NGRP)
        def _(g):
            par = lax.rem(g, 2)
            pltpu.make_async_copy(p_hbm.at[src_v.at[0]],
                                  rows_v.at[par], gsem).wait()
            @pl.when(g >= 1)
            def _():
                pltpu.make_async_copy(rows_v.at[1 - par],
                                      acc.at[dst_v.at[0]], ssem).wait()

            @pl.when(g + 1 < NGRP)
            def _():
                pltpu.async_copy(p_hbm.at[src_v.at[g + 1]],
                                 rows_v.at[1 - par], gsem)

            pltpu.async_copy(rows_v.at[par], acc.at[dst_v.at[g]], ssem,
                             add=True)

        pltpu.make_async_copy(rows_v.at[(NGRP - 1) % 2],
                              acc.at[dst_v.at[0]], ssem).wait()
        plsc.subcore_barrier()
        pltpu.sync_copy(acc.at[pl.ds(s * RPT, RPT)],
                        out_hbm.at[c, pl.ds(s * RPT, RPT)])

    return seg_kernel(p, src2d, dst2d, zrows)


def _proj_kernel(x_ref, w_ref, o_ref):
    o_ref[...] = jnp.dot(x_ref[...], w_ref[...],
                         preferred_element_type=jnp.float32)


def _proj(x, w):
    return pl.pallas_call(
        _proj_kernel,
        out_shape=jax.ShapeDtypeStruct((x.shape[0], w.shape[1]), jnp.float32),
    )(x, w)


def _bn(m, gamma, beta):
    mu = jnp.mean(m, axis=0, keepdims=True)
    var = jnp.mean((m - mu) ** 2, axis=0, keepdims=True)
    return (m - mu) * jax.lax.rsqrt(var + BN_EPS) * gamma + beta


def _mid_kernel(p_ref, agg_ref, b1_ref, w2_ref, b2_ref, g_ref, be_ref,
                w1n_ref, o_ref):
    a = agg_ref[0, :N, :] + agg_ref[1, :N, :]
    m = jax.nn.relu(p_ref[...] + a + b1_ref[...])
    m = jax.nn.relu(jnp.dot(m, w2_ref[...],
                            preferred_element_type=jnp.float32) + b2_ref[...])
    h = _bn(m, g_ref[...], be_ref[...])
    o_ref[...] = jnp.dot(h, w1n_ref[...], preferred_element_type=jnp.float32)


def _mid(p, agg, b1, w2, b2, gamma, beta, w1n):
    return pl.pallas_call(
        _mid_kernel,
        out_shape=jax.ShapeDtypeStruct((N, H), jnp.float32),
    )(p, agg, b1, w2, b2, gamma, beta, w1n)


def _final_kernel(p_ref, agg_ref, b1_ref, w2_ref, b2_ref, g_ref, be_ref,
                  batch_ref, fc1w_ref, fc1b_ref, fc2w_ref, fc2b_ref, o_ref):
    a = agg_ref[0, :N, :] + agg_ref[1, :N, :]
    m = jax.nn.relu(p_ref[...] + a + b1_ref[...])
    m = jax.nn.relu(jnp.dot(m, w2_ref[...],
                            preferred_element_type=jnp.float32) + b2_ref[...])
    h = _bn(m, g_ref[...], be_ref[...])
    gid = lax.broadcasted_iota(jnp.int32, (1, G), 1)
    onehot = (batch_ref[...] == gid).astype(jnp.float32)
    sums = lax.dot_general(onehot, h, (((0,), (0,)), ((), ())),
                           preferred_element_type=jnp.float32)
    cnts = lax.dot_general(onehot, jnp.ones((N, 1), jnp.float32),
                           (((0,), (0,)), ((), ())),
                           preferred_element_type=jnp.float32)
    pooled = sums / jnp.maximum(cnts, 1.0)
    z = jax.nn.relu(jnp.dot(pooled, fc1w_ref[...],
                            preferred_element_type=jnp.float32) + fc1b_ref[...])
    z = jnp.dot(z, fc2w_ref[...],
                preferred_element_type=jnp.float32) + fc2b_ref[...]
    shifted = z - jnp.max(z, axis=-1, keepdims=True)
    o_ref[...] = shifted - jnp.log(
        jnp.sum(jnp.exp(shifted), axis=-1, keepdims=True))


def _final(p, agg, b1, w2, b2, gamma, beta, batch2d, fc1w, fc1b, fc2w, fc2b):
    return pl.pallas_call(
        _final_kernel,
        out_shape=jax.ShapeDtypeStruct((G, C), jnp.float32),
    )(p, agg, b1, w2, b2, gamma, beta, batch2d, fc1w, fc1b, fc2w, fc2b)


def kernel(x, edge_index, batch,
           W1_0, b1_0, W2_0, b2_0, gamma_0, beta_0,
           W1_1, b1_1, W2_1, b2_1, gamma_1, beta_1,
           W1_2, b1_2, W2_2, b2_2, gamma_2, beta_2,
           W1_3, b1_3, W2_3, b2_3, gamma_3, beta_3,
           fc1_W, fc1_b, fc2_W, fc2_b):
    src = edge_index[0]
    dst = edge_index[1]
    pad = E_PAD - E
    grp = GR * CH
    src2d = jnp.concatenate(
        [src, jnp.zeros((pad,), jnp.int32)]).reshape(E_PAD // grp, grp)
    dst2d = jnp.concatenate(
        [dst, jnp.full((pad,), N, jnp.int32)]).reshape(E_PAD // grp, grp)
    zrows = jnp.zeros((N_PAD, H), jnp.float32)
    batch2d = batch.reshape(N, 1)

    b1 = [v.reshape(1, H) for v in (b1_0, b1_1, b1_2, b1_3)]
    b2 = [v.reshape(1, H) for v in (b2_0, b2_1, b2_2, b2_3)]
    gam = [v.reshape(1, H) for v in (gamma_0, gamma_1, gamma_2, gamma_3)]
    bet = [v.reshape(1, H) for v in (beta_0, beta_1, beta_2, beta_3)]
    W1 = [W1_0, W1_1, W1_2, W1_3]
    W2 = [W2_0, W2_1, W2_2, W2_3]

    p = _proj(x, W1[0])
    for l in range(3):
        agg = _segment_sum_sc(p, src2d, dst2d, zrows)
        p = _mid(p, agg, b1[l], W2[l], b2[l], gam[l], bet[l], W1[l + 1])
    agg = _segment_sum_sc(p, src2d, dst2d, zrows)
    return _final(p, agg, b1[3], W2[3], b2[3], gam[3], bet[3], batch2d,
                  fc1_W, fc1_b.reshape(1, H), fc2_W, fc2_b.reshape(1, C))


# speedup vs baseline: 2.2071x; 2.2071x over previous
import functools

import jax
import jax.numpy as jnp
from jax import lax
from jax.experimental import pallas as pl
from jax.experimental.pallas import tpu as pltpu
from jax.experimental.pallas import tpu_sc as plsc

N = 10000
E = 320000
D = 128
H = 32
C = 10
G = 64
BN_EPS = 1e-5

NC, NS = 2, 16
NW = NC * NS
CH = 128
NCH = 80
EW = NCH * CH
E_PAD = NW * EW
N_PAD = 10112
RPT = N_PAD // NS
GR = 5
NGRP = NCH // GR


def _segment_sum_sc(p, src2d, dst2d, zrows):
    mesh = plsc.VectorSubcoreMesh(
        core_axis_name="c", subcore_axis_name="s", num_cores=NC, num_subcores=NS
    )

    @functools.partial(
        pl.kernel,
        out_type=jax.ShapeDtypeStruct((NC, N_PAD, H), jnp.float32),
        mesh=mesh,
        compiler_params=pltpu.CompilerParams(use_tc_tiling_on_sc=False),
        scratch_types=[
            pltpu.VMEM((NGRP, GR * CH), jnp.int32),
            pltpu.VMEM((NGRP, GR * CH), jnp.int32),
            pltpu.VMEM((2, GR * CH, H), jnp.float32),
            pltpu.VMEM_SHARED((N_PAD, H), jnp.float32),
            pltpu.VMEM_SHARED((N_PAD, H), jnp.float32),
            pltpu.SemaphoreType.DMA,
            pltpu.SemaphoreType.DMA,
        ],
    )
    def seg_kernel(p_hbm, src_hbm, dst_hbm, z_hbm, out_hbm,
                   src_v, dst_v, rows_v, acc, p_sp, gsem, ssem):
        c = lax.axis_index("c")
        s = lax.axis_index("s")
        w = c * NS + s
        pltpu.sync_copy(z_hbm.at[pl.ds(s * RPT, RPT)],
                        acc.at[pl.ds(s * RPT, RPT)])
        pltpu.sync_copy(p_hbm.at[pl.ds(s * RPT, RPT)],
                        p_sp.at[pl.ds(s * RPT, RPT)])
        pltpu.sync_copy(src_hbm.at[pl.ds(w * NGRP, NGRP)], src_v)
        pltpu.sync_copy(dst_hbm.at[pl.ds(w * NGRP, NGRP)], dst_v)
        plsc.subcore_barrier()

        pltpu.async_copy(p_sp.at[src_v.at[0]], rows_v.at[0], gsem)

        @pl.loop(0, NGRP)
        def _(g):
            par = lax.rem(g, 2)
            pltpu.make_async_copy(p_sp.at[src_v.at[0]],
                                  rows_v.at[par], gsem).wait()
            @pl.when(g >= 1)
            def _():
                pltpu.make_async_copy(rows_v.at[1 - par],
                                      acc.at[dst_v.at[0]], ssem).wait()

            @pl.when(g + 1 < NGRP)
            def _():
                pltpu.async_copy(p_sp.at[src_v.at[g + 1]],
                                 rows_v.at[1 - par], gsem)

            pltpu.async_copy(rows_v.at[par], acc.at[dst_v.at[g]], ssem,
                             add=True)

        pltpu.make_async_copy(rows_v.at[(NGRP - 1) % 2],
                              acc.at[dst_v.at[0]], ssem).wait()
        plsc.subcore_barrier()
        pltpu.sync_copy(acc.at[pl.ds(s * RPT, RPT)],
                        out_hbm.at[c, pl.ds(s * RPT, RPT)])

    return seg_kernel(p, src2d, dst2d, zrows)


def _proj_kernel(x_ref, w_ref, o_ref):
    o_ref[:N, :] = jnp.dot(x_ref[...], w_ref[...],
                           preferred_element_type=jnp.float32)
    o_ref[N:, :] = jnp.zeros((N_PAD - N, H), jnp.float32)


def _proj(x, w):
    return pl.pallas_call(
        _proj_kernel,
        out_shape=jax.ShapeDtypeStruct((N_PAD, w.shape[1]), jnp.float32),
    )(x, w)


def _bn(m, gamma, beta):
    mu = jnp.mean(m, axis=0, keepdims=True)
    var = jnp.mean((m - mu) ** 2, axis=0, keepdims=True)
    return (m - mu) * jax.lax.rsqrt(var + BN_EPS) * gamma + beta


def _mid_kernel(p_ref, agg_ref, b1_ref, w2_ref, b2_ref, g_ref, be_ref,
                w1n_ref, o_ref):
    a = agg_ref[0, :N, :] + agg_ref[1, :N, :]
    m = jax.nn.relu(p_ref[:N, :] + a + b1_ref[...])
    m = jax.nn.relu(jnp.dot(m, w2_ref[...],
                            preferred_element_type=jnp.float32) + b2_ref[...])
    h = _bn(m, g_ref[...], be_ref[...])
    o_ref[:N, :] = jnp.dot(h, w1n_ref[...], preferred_element_type=jnp.float32)
    o_ref[N:, :] = jnp.zeros((N_PAD - N, H), jnp.float32)


def _mid(p, agg, b1, w2, b2, gamma, beta, w1n):
    return pl.pallas_call(
        _mid_kernel,
        out_shape=jax.ShapeDtypeStruct((N_PAD, H), jnp.float32),
    )(p, agg, b1, w2, b2, gamma, beta, w1n)


def _final_kernel(p_ref, agg_ref, b1_ref, w2_ref, b2_ref, g_ref, be_ref,
                  batch_ref, fc1w_ref, fc1b_ref, fc2w_ref, fc2b_ref, o_ref):
    a = agg_ref[0, :N, :] + agg_ref[1, :N, :]
    m = jax.nn.relu(p_ref[:N, :] + a + b1_ref[...])
    m = jax.nn.relu(jnp.dot(m, w2_ref[...],
                            preferred_element_type=jnp.float32) + b2_ref[...])
    h = _bn(m, g_ref[...], be_ref[...])
    gid = lax.broadcasted_iota(jnp.int32, (1, G), 1)
    onehot = (batch_ref[...] == gid).astype(jnp.float32)
    sums = lax.dot_general(onehot, h, (((0,), (0,)), ((), ())),
                           preferred_element_type=jnp.float32)
    cnts = lax.dot_general(onehot, jnp.ones((N, 1), jnp.float32),
                           (((0,), (0,)), ((), ())),
                           preferred_element_type=jnp.float32)
    pooled = sums / jnp.maximum(cnts, 1.0)
    z = jax.nn.relu(jnp.dot(pooled, fc1w_ref[...],
                            preferred_element_type=jnp.float32) + fc1b_ref[...])
    z = jnp.dot(z, fc2w_ref[...],
                preferred_element_type=jnp.float32) + fc2b_ref[...]
    shifted = z - jnp.max(z, axis=-1, keepdims=True)
    o_ref[...] = shifted - jnp.log(
        jnp.sum(jnp.exp(shifted), axis=-1, keepdims=True))


def _final(p, agg, b1, w2, b2, gamma, beta, batch2d, fc1w, fc1b, fc2w, fc2b):
    return pl.pallas_call(
        _final_kernel,
        out_shape=jax.ShapeDtypeStruct((G, C), jnp.float32),
    )(p, agg, b1, w2, b2, gamma, beta, batch2d, fc1w, fc1b, fc2w, fc2b)


def kernel(x, edge_index, batch,
           W1_0, b1_0, W2_0, b2_0, gamma_0, beta_0,
           W1_1, b1_1, W2_1, b2_1, gamma_1, beta_1,
           W1_2, b1_2, W2_2, b2_2, gamma_2, beta_2,
           W1_3, b1_3, W2_3, b2_3, gamma_3, beta_3,
           fc1_W, fc1_b, fc2_W, fc2_b):
    src = edge_index[0]
    dst = edge_index[1]
    pad = E_PAD - E
    grp = GR * CH
    src2d = jnp.concatenate(
        [src, jnp.zeros((pad,), jnp.int32)]).reshape(E_PAD // grp, grp)
    dst2d = jnp.concatenate(
        [dst, jnp.full((pad,), N, jnp.int32)]).reshape(E_PAD // grp, grp)
    zrows = jnp.zeros((N_PAD, H), jnp.float32)
    batch2d = batch.reshape(N, 1)

    b1 = [v.reshape(1, H) for v in (b1_0, b1_1, b1_2, b1_3)]
    b2 = [v.reshape(1, H) for v in (b2_0, b2_1, b2_2, b2_3)]
    gam = [v.reshape(1, H) for v in (gamma_0, gamma_1, gamma_2, gamma_3)]
    bet = [v.reshape(1, H) for v in (beta_0, beta_1, beta_2, beta_3)]
    W1 = [W1_0, W1_1, W1_2, W1_3]
    W2 = [W2_0, W2_1, W2_2, W2_3]

    p = _proj(x, W1[0])
    for l in range(3):
        agg = _segment_sum_sc(p, src2d, dst2d, zrows)
        p = _mid(p, agg, b1[l], W2[l], b2[l], gam[l], bet[l], W1[l + 1])
    agg = _segment_sum_sc(p, src2d, dst2d, zrows)
    return _final(p, agg, b1[3], W2[3], b2[3], gam[3], bet[3], batch2d,
                  fc1_W, fc1_b.reshape(1, H), fc2_W, fc2_b.reshape(1, C))
